# baseline (speedup 1.0000x reference)
import functools

import jax
import jax.numpy as jnp
from jax import lax
from jax.experimental import pallas as pl
from jax.experimental.pallas import tpu as pltpu
from jax.experimental.pallas import tpu_sc as plsc

LANES = 16


def _build(B, T, V, D):
    info = plsc.get_sparse_core_info()
    NW = info.num_cores * info.num_subcores
    CH = T // NW
    mesh = plsc.VectorSubcoreMesh(core_axis_name="c", subcore_axis_name="s")

    @functools.partial(
        pl.kernel,
        out_type=jax.ShapeDtypeStruct((B * T, D), jnp.float32),
        mesh=mesh,
        scratch_types=[
            pltpu.VMEM((B, CH), jnp.int32),
            pltpu.VMEM((CH, D), jnp.float32),
            pltpu.VMEM((4, CH, D), jnp.float32),
            pltpu.SemaphoreType.DMA,
            pltpu.SemaphoreType.DMA,
            pltpu.SemaphoreType.DMA,
            pltpu.SemaphoreType.DMA,
            pltpu.SemaphoreType.DMA,
            pltpu.SemaphoreType.DMA,
            pltpu.SemaphoreType.DMA,
            pltpu.SemaphoreType.DMA,
        ],
    )
    def run(idx_hbm, tok_hbm, pos_hbm, out_hbm, idx_v, pos_v, buf,
            gsem0, gsem1, gsem2, gsem3, wsem0, wsem1, wsem2, wsem3):
        w = lax.axis_index("s") * info.num_cores + lax.axis_index("c")
        t0 = w * CH
        gsems = (gsem0, gsem1, gsem2, gsem3)
        wsems = (wsem0, wsem1, wsem2, wsem3)

        def fire_gather(b, s):
            pltpu.async_copy(tok_hbm.at[idx_v.at[b]], buf.at[s], gsems[s])

        def drain_gather(b, s):
            pltpu.make_async_copy(
                tok_hbm.at[idx_v.at[b]], buf.at[s], gsems[s]).wait()

        def fire_write(b, s):
            pltpu.async_copy(
                buf.at[s], out_hbm.at[pl.ds(b * T + t0, CH)], wsems[s])

        def drain_write(b, s):
            pltpu.make_async_copy(
                buf.at[s], out_hbm.at[pl.ds(b * T + t0, CH)], wsems[s]).wait()

        pltpu.sync_copy(idx_hbm.at[w], idx_v)
        fire_gather(0, 0)
        fire_gather(1, 1)
        pltpu.sync_copy(pos_hbm.at[pl.ds(t0, CH)], pos_v)

        def body(i, carry):
            for u in range(4):
                b = 4 * i + u
                s2 = (u + 2) % 4

                @pl.when(b >= 2)
                def _():
                    drain_write(b - 2, s2)

                @pl.when(b + 2 < B)
                def _():
                    fire_gather(b + 2, s2)

                drain_gather(b, u)

                def add_row(r, c):
                    for k in range(D // LANES):
                        sl = pl.ds(k * LANES, LANES)
                        buf[u, r, sl] = buf[u, r, sl] + pos_v[r, sl]
                    return c

                lax.fori_loop(0, CH, add_row, 0)
                fire_write(b, u)

            return carry

        lax.fori_loop(0, B // 4, body, 0)
        drain_write(B - 2, 2)
        drain_write(B - 1, 3)

    return run


def kernel(idx, tok_emb, pos_emb):
    B, T = idx.shape
    V, D = tok_emb.shape
    info = plsc.get_sparse_core_info()
    NW = info.num_cores * info.num_subcores
    CH = T // NW
    idx_r = idx.astype(jnp.int32).reshape(B, NW, CH).transpose(1, 0, 2)
    run = _build(B, T, V, D)
    out = run(idx_r, tok_emb, pos_emb)
    return out.reshape(B, T, D)


# speedup vs baseline: 1.9405x; 1.9405x over previous
import functools

import jax
import jax.numpy as jnp
from jax import lax
from jax.experimental import pallas as pl
from jax.experimental.pallas import tpu as pltpu
from jax.experimental.pallas import tpu_sc as plsc

LANES = 16


def _build(B, T, V, D):
    info = plsc.get_sparse_core_info()
    NW = info.num_cores * info.num_subcores
    CH = T // NW
    mesh = plsc.VectorSubcoreMesh(core_axis_name="c", subcore_axis_name="s")

    @functools.partial(
        pl.kernel,
        out_type=jax.ShapeDtypeStruct((B * T, D), jnp.float32),
        mesh=mesh,
        scratch_types=[
            pltpu.VMEM((B, CH), jnp.int32),
            pltpu.VMEM((CH, D), jnp.float32),
            pltpu.VMEM((CH, D), jnp.float32),
            pltpu.VMEM((CH, D), jnp.float32),
            pltpu.VMEM((CH, D), jnp.float32),
            pltpu.VMEM((CH, D), jnp.float32),
            pltpu.SemaphoreType.DMA,
            pltpu.SemaphoreType.DMA,
            pltpu.SemaphoreType.DMA,
            pltpu.SemaphoreType.DMA,
            pltpu.SemaphoreType.DMA,
            pltpu.SemaphoreType.DMA,
            pltpu.SemaphoreType.DMA,
            pltpu.SemaphoreType.DMA,
        ],
    )
    def run(idx_hbm, tok_hbm, pos_hbm, out_hbm, idx_v, pos_v,
            buf0, buf1, buf2, buf3,
            gsem0, gsem1, gsem2, gsem3, wsem0, wsem1, wsem2, wsem3):
        bufs = (buf0, buf1, buf2, buf3)
        w = lax.axis_index("s") * info.num_cores + lax.axis_index("c")
        t0 = w * CH
        gsems = (gsem0, gsem1, gsem2, gsem3)
        wsems = (wsem0, wsem1, wsem2, wsem3)

        def fire_gather(b, s):
            pltpu.async_copy(tok_hbm.at[idx_v.at[b]], bufs[s], gsems[s])

        def drain_gather(b, s):
            pltpu.make_async_copy(
                tok_hbm.at[idx_v.at[b]], bufs[s], gsems[s]).wait()

        def fire_write(b, s):
            pltpu.async_copy(
                bufs[s], out_hbm.at[pl.ds(b * T + t0, CH)], wsems[s])

        def drain_write(b, s):
            pltpu.make_async_copy(
                bufs[s], out_hbm.at[pl.ds(b * T + t0, CH)], wsems[s]).wait()

        pltpu.sync_copy(idx_hbm.at[w], idx_v)
        fire_gather(0, 0)
        fire_gather(1, 1)
        pltpu.sync_copy(pos_hbm.at[pl.ds(t0, CH)], pos_v)

        def body(i, carry):
            for u in range(4):
                b = 4 * i + u
                s2 = (u + 2) % 4

                @pl.when(b >= 2)
                def _():
                    drain_write(b - 2, s2)

                @pl.when(b + 2 < B)
                def _():
                    fire_gather(b + 2, s2)

                drain_gather(b, u)

                def add_row(r, c):
                    for k in range(D // LANES):
                        sl = pl.ds(k * LANES, LANES)
                        bufs[u][r, sl] = bufs[u][r, sl] + pos_v[r, sl]
                    return c

                lax.fori_loop(0, CH, add_row, 0)
                fire_write(b, u)

            return carry

        lax.fori_loop(0, B // 4, body, 0)
        drain_write(B - 2, 2)
        drain_write(B - 1, 3)

    return run


def kernel(idx, tok_emb, pos_emb):
    B, T = idx.shape
    V, D = tok_emb.shape
    info = plsc.get_sparse_core_info()
    NW = info.num_cores * info.num_subcores
    CH = T // NW
    idx_r = idx.astype(jnp.int32).reshape(B, NW, CH).transpose(1, 0, 2)
    run = _build(B, T, V, D)
    out = run(idx_r, tok_emb, pos_emb)
    return out.reshape(B, T, D)
